# Initial kernel scaffold; baseline (speedup 1.0000x reference)
import functools

import jax
import jax.numpy as jnp
from jax import lax
from jax.experimental import pallas as pl
from jax.experimental.pallas import tpu as pltpu
from jax.experimental.pallas import tpu_sc as plsc

VOCAB = 1000000
EMB = 32
BATCH = 16384
HIST = 50

NC = 2
NS = 16
NW = NC * NS

B = BATCH * HIST
C = 128
NROWS = B // C
CHUNKS_PER_W = NROWS // NW


def _make_kernel():
  mesh = plsc.VectorSubcoreMesh(
      core_axis_name="c", subcore_axis_name="s", num_cores=NC, num_subcores=NS
  )

  @functools.partial(
      pl.kernel,
      out_type=jax.ShapeDtypeStruct((B, EMB), jnp.float32),
      mesh=mesh,
      scratch_types=[
          pltpu.VMEM((CHUNKS_PER_W, C), jnp.int32),
          pltpu.VMEM((C, EMB), jnp.float32),
          pltpu.SemaphoreType.DMA,
      ],
  )
  def gather_kernel(x_hbm, w_hbm, out_hbm, idx_v, rows_v, sem):
    wid = lax.axis_index("s") * NC + lax.axis_index("c")
    row0 = wid * CHUNKS_PER_W
    pltpu.sync_copy(x_hbm.at[pl.ds(row0, CHUNKS_PER_W)], idx_v)

    @pl.loop(0, CHUNKS_PER_W)
    def _(j):
      pltpu.async_copy(w_hbm.at[idx_v.at[j]], rows_v, sem).wait()
      pltpu.sync_copy(rows_v, out_hbm.at[pl.ds((row0 + j) * C, C)])

  return gather_kernel


_kernel_call = _make_kernel()


@jax.jit
def kernel(x, W):
  idx = x.astype(jnp.int32).reshape(NROWS, C)
  out = _kernel_call(idx, W)
  return out.reshape(BATCH, HIST, EMB)


# speedup vs baseline: 1.0235x; 1.0235x over previous
import functools

import jax
import jax.numpy as jnp
from jax import lax
from jax.experimental import pallas as pl
from jax.experimental.pallas import tpu as pltpu
from jax.experimental.pallas import tpu_sc as plsc

VOCAB = 1000000
EMB = 32
BATCH = 16384
HIST = 50

NC = 2
NS = 16
NW = NC * NS

B = BATCH * HIST
C = 128
NROWS = B // C
CHUNKS_PER_W = NROWS // NW


def _make_kernel():
  mesh = plsc.VectorSubcoreMesh(
      core_axis_name="c", subcore_axis_name="s", num_cores=NC, num_subcores=NS
  )

  @functools.partial(
      pl.kernel,
      out_type=jax.ShapeDtypeStruct((B, EMB), jnp.float32),
      mesh=mesh,
      scratch_types=[
          pltpu.VMEM((CHUNKS_PER_W, C), jnp.int32),
          pltpu.VMEM((C, EMB), jnp.float32),
          pltpu.SemaphoreType.DMA,
      ],
      compiler_params=pltpu.CompilerParams(use_tc_tiling_on_sc=False),
  )
  def gather_kernel(x_hbm, w_hbm, out_hbm, idx_v, rows_v, sem):
    wid = lax.axis_index("s") * NC + lax.axis_index("c")
    row0 = wid * CHUNKS_PER_W
    pltpu.sync_copy(x_hbm.at[pl.ds(row0, CHUNKS_PER_W)], idx_v)

    @pl.loop(0, CHUNKS_PER_W)
    def _(j):
      pltpu.async_copy(w_hbm.at[idx_v.at[j]], rows_v, sem).wait()
      pltpu.sync_copy(rows_v, out_hbm.at[pl.ds((row0 + j) * C, C)])

  return gather_kernel


_kernel_call = _make_kernel()


@jax.jit
def kernel(x, W):
  idx = x.astype(jnp.int32).reshape(NROWS, C)
  out = _kernel_call(idx, W)
  return out.reshape(BATCH, HIST, EMB)


# speedup vs baseline: 1.1137x; 1.0881x over previous
import functools

import jax
import jax.numpy as jnp
from jax import lax
from jax.experimental import pallas as pl
from jax.experimental.pallas import tpu as pltpu
from jax.experimental.pallas import tpu_sc as plsc

VOCAB = 1000000
EMB = 32
BATCH = 16384
HIST = 50

NC = 2
NS = 16
NW = NC * NS

B = BATCH * HIST
C = 128
NROWS = B // C
CHUNKS_PER_W = NROWS // NW

GROUP = 4
ROWS_G = GROUP * C
NGROUP = CHUNKS_PER_W // GROUP
NBUF = 4


def _make_kernel():
  mesh = plsc.VectorSubcoreMesh(
      core_axis_name="c", subcore_axis_name="s", num_cores=NC, num_subcores=NS
  )

  @functools.partial(
      pl.kernel,
      out_type=jax.ShapeDtypeStruct((B, EMB), jnp.float32),
      mesh=mesh,
      scratch_types=[
          pltpu.VMEM((CHUNKS_PER_W, C), jnp.int32),
          [pltpu.VMEM((ROWS_G, EMB), jnp.float32) for _ in range(NBUF)],
          [pltpu.SemaphoreType.DMA for _ in range(NBUF)],
          [pltpu.SemaphoreType.DMA for _ in range(NBUF)],
      ],
      compiler_params=pltpu.CompilerParams(use_tc_tiling_on_sc=False),
  )
  def gather_kernel(x_hbm, w_hbm, out_hbm, idx_v, bufs, gsems, wsems):
    wid = lax.axis_index("s") * NC + lax.axis_index("c")
    row0 = wid * CHUNKS_PER_W
    out0 = row0 * C

    pltpu.sync_copy(x_hbm.at[pl.ds(row0, CHUNKS_PER_W)], idx_v)

    def start_g(g, b):
      for k in range(GROUP):
        pltpu.async_copy(
            w_hbm.at[idx_v.at[g * GROUP + k]],
            bufs[b].at[pl.ds(k * C, C)],
            gsems[b],
        )

    def wait_g(b):
      pltpu.make_async_copy(
          w_hbm.at[pl.ds(0, ROWS_G)], bufs[b], gsems[b]
      ).wait()

    def start_w(g, b):
      pltpu.async_copy(
          bufs[b], out_hbm.at[pl.ds(out0 + g * ROWS_G, ROWS_G)], wsems[b]
      )

    def wait_w(g, b):
      pltpu.make_async_copy(
          bufs[b], out_hbm.at[pl.ds(out0 + g * ROWS_G, ROWS_G)], wsems[b]
      ).wait()

    start_g(0, 0)
    start_g(1, 1)
    start_g(2, 2)
    wait_g(0)
    start_w(0, 0)
    start_g(3, 3)
    wait_g(1)
    start_w(1, 1)

    @pl.loop(4, NGROUP - NGROUP % NBUF, step=NBUF)
    def _(g0):
      for j in range(NBUF):
        g = g0 + j
        b = (4 + j) % NBUF
        b2 = (2 + j) % NBUF
        wait_w(g - NBUF, b)
        start_g(g, b)
        wait_g(b2)
        start_w(g - 2, b2)

    wait_w(44, 0)
    start_g(48, 0)
    wait_g(2)
    start_w(46, 2)
    wait_w(45, 1)
    start_g(49, 1)
    wait_g(3)
    start_w(47, 3)
    wait_g(0)
    start_w(48, 0)
    wait_g(1)
    start_w(49, 1)
    wait_w(46, 2)
    wait_w(47, 3)
    wait_w(48, 0)
    wait_w(49, 1)

  return gather_kernel


_kernel_call = _make_kernel()


@jax.jit
def kernel(x, W):
  idx = x.astype(jnp.int32).reshape(NROWS, C)
  out = _kernel_call(idx, W)
  return out.reshape(BATCH, HIST, EMB)


# speedup vs baseline: 1.8065x; 1.6221x over previous
import functools

import jax
import jax.numpy as jnp
from jax import lax
from jax.experimental import pallas as pl
from jax.experimental.pallas import tpu as pltpu
from jax.experimental.pallas import tpu_sc as plsc

VOCAB = 1000000
EMB = 32
BATCH = 16384
HIST = 50

NC = 2
NS = 16
NW = NC * NS

BAT_PER_W = BATCH // NW
GB = 16
NGROUP = BAT_PER_W // GB
NBUF = 3


def _make_kernel():
  mesh = plsc.VectorSubcoreMesh(
      core_axis_name="c", subcore_axis_name="s", num_cores=NC, num_subcores=NS
  )

  @functools.partial(
      pl.kernel,
      out_type=jax.ShapeDtypeStruct((BATCH, HIST, EMB), jnp.float32),
      mesh=mesh,
      scratch_types=[
          pltpu.VMEM((BAT_PER_W, HIST), jnp.int32),
          [pltpu.VMEM((GB, HIST, EMB), jnp.float32) for _ in range(NBUF)],
          [pltpu.SemaphoreType.DMA for _ in range(NBUF)],
          [pltpu.SemaphoreType.DMA for _ in range(NBUF)],
      ],
      compiler_params=pltpu.CompilerParams(use_tc_tiling_on_sc=False),
  )
  def gather_kernel(x_hbm, w_hbm, out_hbm, idx_v, bufs, gsems, wsems):
    wid = lax.axis_index("s") * NC + lax.axis_index("c")
    bat0 = wid * BAT_PER_W

    pltpu.sync_copy(x_hbm.at[pl.ds(bat0, BAT_PER_W)], idx_v)

    def start_g(g, b):
      for k in range(GB):
        pltpu.async_copy(
            w_hbm.at[idx_v.at[g * GB + k]], bufs[b].at[k], gsems[b]
        )

    def wait_g(b):
      pltpu.make_async_copy(
          out_hbm.at[pl.ds(0, GB)], bufs[b], gsems[b]
      ).wait()

    def start_w(g, b):
      pltpu.async_copy(
          bufs[b], out_hbm.at[pl.ds(bat0 + g * GB, GB)], wsems[b]
      )

    def wait_w(g, b):
      pltpu.make_async_copy(
          bufs[b], out_hbm.at[pl.ds(bat0 + g * GB, GB)], wsems[b]
      ).wait()

    start_g(0, 0)
    start_g(1, 1)
    start_g(2, 2)
    wait_g(0)
    start_w(0, 0)

    @pl.loop(3, 30, step=NBUF)
    def _(g0):
      for j in range(NBUF):
        g = g0 + j
        b = j
        b2 = (j + 1) % NBUF
        wait_w(g - NBUF, b)
        start_g(g, b)
        wait_g(b2)
        start_w(g - 2, b2)

    wait_w(27, 0)
    start_g(30, 0)
    wait_g(1)
    start_w(28, 1)
    wait_w(28, 1)
    start_g(31, 1)
    wait_g(2)
    start_w(29, 2)
    wait_g(0)
    start_w(30, 0)
    wait_g(1)
    start_w(31, 1)
    wait_w(29, 2)
    wait_w(30, 0)
    wait_w(31, 1)

  return gather_kernel


_kernel_call = _make_kernel()


@jax.jit
def kernel(x, W):
  return _kernel_call(x.astype(jnp.int32), W)
